# baseline (speedup 1.0000x reference)
import functools

import jax
import jax.numpy as jnp
from jax import lax
from jax.experimental import pallas as pl
from jax.experimental.pallas import tpu as pltpu

OBS_DIM = 512
ACTION_DIM = 64
N_AGENTS = 8
HIDDEN_DIM = 1024
ASP_DIM = 256
ASP_HIDDEN = 512

BM = 256
OUT_COLS = 80


def _fused_body(ba_ref, x_ref, W1_ref, b1_ref, W2_ref, b2_ref,
                Wa1_ref, ba1_ref, Wa2_ref, ba2_ref,
                Wv_ref, bv_ref, Wp1_ref, bp1_ref, Wp2_ref, bp2_ref,
                out_ref):
    f32 = jnp.float32
    x = x_ref[...]
    h1 = jnp.maximum(jnp.dot(x, W1_ref[...], preferred_element_type=f32) + b1_ref[...], 0.0)
    h = jnp.maximum(jnp.dot(h1, W2_ref[...], preferred_element_type=f32) + b2_ref[...], 0.0)
    a1 = jnp.maximum(jnp.dot(h, Wa1_ref[0], preferred_element_type=f32) + ba1_ref[0], 0.0)
    f = jnp.dot(a1, Wa2_ref[0], preferred_element_type=f32) + ba2_ref[0]
    p1 = jnp.maximum(
        jnp.dot(h, Wp1_ref[:HIDDEN_DIM, :], preferred_element_type=f32)
        + jnp.dot(f, Wp1_ref[HIDDEN_DIM:, :], preferred_element_type=f32)
        + bp1_ref[...], 0.0)
    logits = jnp.dot(p1, Wp2_ref[...], preferred_element_type=f32) + bp2_ref[...]
    value = (jnp.sum(h * Wv_ref[:, :HIDDEN_DIM], axis=1, keepdims=True)
             + jnp.sum(f * Wv_ref[:, HIDDEN_DIM:], axis=1, keepdims=True)
             + bv_ref[0])
    out_ref[...] = jnp.concatenate(
        [logits, jnp.broadcast_to(value, (value.shape[0], OUT_COLS - ACTION_DIM))], axis=1)


def _fused_net(x_sorted, block_agent, W1, b1, W2, b2, Wa1, ba1, Wa2, ba2,
               Wv, bv, Wp1, bp1, Wp2, bp2, *, interpret=False):
    m_pad = x_sorted.shape[0]
    nb = m_pad // BM
    grid_spec = pltpu.PrefetchScalarGridSpec(
        num_scalar_prefetch=1,
        grid=(nb,),
        in_specs=[
            pl.BlockSpec((BM, OBS_DIM), lambda i, ba: (i, 0)),
            pl.BlockSpec((OBS_DIM, HIDDEN_DIM), lambda i, ba: (0, 0)),
            pl.BlockSpec((1, HIDDEN_DIM), lambda i, ba: (0, 0)),
            pl.BlockSpec((HIDDEN_DIM, HIDDEN_DIM), lambda i, ba: (0, 0)),
            pl.BlockSpec((1, HIDDEN_DIM), lambda i, ba: (0, 0)),
            pl.BlockSpec((1, HIDDEN_DIM, ASP_HIDDEN), lambda i, ba: (ba[i], 0, 0)),
            pl.BlockSpec((1, 1, ASP_HIDDEN), lambda i, ba: (ba[i], 0, 0)),
            pl.BlockSpec((1, ASP_HIDDEN, ASP_DIM), lambda i, ba: (ba[i], 0, 0)),
            pl.BlockSpec((1, 1, ASP_DIM), lambda i, ba: (ba[i], 0, 0)),
            pl.BlockSpec((1, HIDDEN_DIM + ASP_DIM), lambda i, ba: (0, 0)),
            pl.BlockSpec(memory_space=pltpu.SMEM),
            pl.BlockSpec((HIDDEN_DIM + ASP_DIM, HIDDEN_DIM), lambda i, ba: (0, 0)),
            pl.BlockSpec((1, HIDDEN_DIM), lambda i, ba: (0, 0)),
            pl.BlockSpec((HIDDEN_DIM, ACTION_DIM), lambda i, ba: (0, 0)),
            pl.BlockSpec((1, ACTION_DIM), lambda i, ba: (0, 0)),
        ],
        out_specs=pl.BlockSpec((BM, OUT_COLS), lambda i, ba: (i, 0)),
    )
    return pl.pallas_call(
        _fused_body,
        grid_spec=grid_spec,
        out_shape=jax.ShapeDtypeStruct((m_pad, OUT_COLS), jnp.float32),
        interpret=interpret,
    )(block_agent, x_sorted,
      W1, b1.reshape(1, -1), W2, b2.reshape(1, -1),
      Wa1, ba1.reshape(N_AGENTS, 1, ASP_HIDDEN), Wa2, ba2.reshape(N_AGENTS, 1, ASP_DIM),
      Wv.reshape(1, -1), bv, Wp1, bp1.reshape(1, -1), Wp2, bp2.reshape(1, -1))


def _routing(ids, m_pad):
    m = ids.shape[0]
    onehot = (ids[:, None] == jnp.arange(N_AGENTS, dtype=ids.dtype)[None, :]).astype(jnp.int32)
    cum = jnp.cumsum(onehot, axis=0)
    rank = jnp.take_along_axis(cum, ids[:, None].astype(jnp.int32), axis=1)[:, 0] - 1
    counts = cum[-1]
    padded = ((counts + BM - 1) // BM) * BM
    ends = jnp.cumsum(padded)
    offs = ends - padded
    dest = offs[ids] + rank
    tok_at = jnp.zeros((m_pad,), jnp.int32).at[dest].set(jnp.arange(m, dtype=jnp.int32))
    nb = m_pad // BM
    block_start = jnp.arange(nb, dtype=jnp.int32) * BM
    block_agent = jnp.minimum(
        jnp.searchsorted(ends, block_start, side='right').astype(jnp.int32), N_AGENTS - 1)
    return tok_at, dest, block_agent


def kernel(obs, agent_ids, W1, b1, W2, b2, Wa1, ba1, Wa2, ba2, Wv, bv, Wp1, bp1, Wp2, bp2):
    b, n, o = obs.shape
    m = b * n
    m_pad = m + N_AGENTS * BM
    x = obs.reshape(m, o)
    ids = agent_ids.reshape(m).astype(jnp.int32)

    tok_at, dest, block_agent = _routing(ids, m_pad)

    x_sorted = jnp.take(x, tok_at, axis=0)
    outbuf = _fused_net(x_sorted, block_agent, W1, b1, W2, b2, Wa1, ba1, Wa2, ba2,
                        Wv, bv, Wp1, bp1, Wp2, bp2)
    out = jnp.take(outbuf, dest, axis=0)

    values = out[:, ACTION_DIM].reshape(b, n)
    logits = out[:, :ACTION_DIM].reshape(b, n, ACTION_DIM)
    return (values, logits)
